# baseline (speedup 1.0000x reference)
import jax
import jax.numpy as jnp
from jax import lax
from jax.experimental import pallas as pl
from jax.experimental.pallas import tpu as pltpu
from jax.experimental.pallas import tpu_sc as plsc

_K = 64
_NUM_SAMPLES = 4
_B = 128
_N = 32768
_ROW_BLOCK = 8

_INT_MIN = -(2 ** 31)
_NW = 32
_RPW = _B // _NW
_NBLK = 32
_BLKV = 64
_NGRP = _NBLK * 16
_POOL = 4096


def _monokey(x):
    b = jax.lax.bitcast_convert_type(x, jnp.int32)
    return b ^ ((b >> 31) & jnp.int32(0x7FFFFFFF))


def _monokey_i(b):
    return b ^ ((b >> 31) & jnp.int32(0x7FFFFFFF))


def _count16(mask):
    return plsc.cumsum(mask.astype(jnp.int32))[15]


def _radix_select(ref, nv, target, start_bit, nbits, init_prefix):
    def bitbody(i, prefix_u):
        b = start_bit - i
        cand_u = prefix_u | lax.shift_left(jnp.int32(1), b)
        cand_s = cand_u ^ _INT_MIN

        def cntbody(v, acc):
            kk = ref[pl.ds(v * 16, 16)]
            return acc + _count16(kk >= cand_s)

        cnt = lax.fori_loop(0, nv, cntbody, jnp.int32(0))
        return jnp.where(cnt >= target, cand_u, prefix_u)

    prefix_u = lax.fori_loop(0, nbits, bitbody, init_prefix)
    return prefix_u ^ _INT_MIN


def _sc_select(x_i32, nrows):
    rpw = nrows // _NW
    nst = rpw // 2
    mesh = plsc.VectorSubcoreMesh(
        core_axis_name="c", subcore_axis_name="s", num_cores=2, num_subcores=16
    )

    def body(x_hbm, tkey_hbm, ithr_hbm, row_v, kmax_v, qids_v, poolk_v,
             pooli_v, keys2_v, stage_v):
        wid = lax.axis_index("c") * 16 + lax.axis_index("s")
        iota16 = lax.iota(jnp.int32, 16)
        iota16x = iota16 * 16

        def do_row(row):
            pltpu.sync_copy(x_hbm.at[row], row_v)

            def blkbody(ob, unused):
                base0 = ob * 1024

                def inner(it, m):
                    base = base0 + it * 128
                    for jj in range(8):
                        m = jnp.maximum(
                            m, _monokey_i(row_v[pl.ds(base + jj * 16, 16)])
                        )
                    return m

                m = lax.fori_loop(
                    0, 8, inner, jnp.full((16,), _INT_MIN, jnp.int32)
                )
                kmax_v[pl.ds(ob * 16, 16)] = m
                return unused

            lax.fori_loop(0, _NBLK, blkbody, jnp.int32(0))

            tau_s = _radix_select(kmax_v, _NGRP // 16, _K, 31, 16, jnp.int32(0))

            def qbody(i, qoff):
                kk = kmax_v[pl.ds(i * 16, 16)]
                m = kk >= tau_s
                ci = plsc.cumsum(m.astype(jnp.int32))
                pos = qoff + ci - 1
                plsc.store_scatter(qids_v, [pos], i * 16 + iota16, mask=m)
                return qoff + ci[15]

            nq = lax.fori_loop(0, _NGRP // 16, qbody, jnp.int32(0))

            def gbody(q, poff):
                gid = qids_v[pl.ds(q, 16)][0]
                base = (gid >> 4) * 1024 + (gid & 15)
                offs = jnp.int32(0)
                for jj in range(4):
                    idx = base + jj * 256 + iota16x
                    kv = _monokey_i(plsc.load_gather(row_v, [idx]))
                    m = kv >= tau_s
                    ci = plsc.cumsum(m.astype(jnp.int32))
                    pos = jnp.minimum(poff + offs + ci - 1,
                                      jnp.int32(_POOL - 1))
                    plsc.store_scatter(poolk_v, [pos], kv, mask=m)
                    plsc.store_scatter(pooli_v, [pos], idx, mask=m)
                    offs = offs + ci[15]
                return jnp.minimum(poff + offs, jnp.int32(_POOL - 16))

            poff = lax.fori_loop(0, nq, gbody, jnp.int32(0))

            plsc.store_scatter(
                poolk_v, [poff + iota16],
                jnp.full((16,), _INT_MIN, jnp.int32),
            )

            nv = (poff + 15) >> 4
            t_s = _radix_select(poolk_v, nv, _K, 31, 32, jnp.int32(0))

            def ebody(v, acc):
                kk = poolk_v[pl.ds(v * 16, 16)]
                acc = acc + _count16(kk > t_s)
                k2 = jnp.where(
                    kk == t_s, jnp.int32(_N - 1) - pooli_v[pl.ds(v * 16, 16)],
                    _INT_MIN,
                )
                keys2_v[pl.ds(v * 16, 16)] = k2
                return acc

            cnt_gt = lax.fori_loop(0, nv, ebody, jnp.int32(0))
            need = _K - cnt_gt
            v2_s = _radix_select(keys2_v, nv, need, 14, 15, jnp.int32(_INT_MIN))
            ithr = jnp.int32(_N - 1) - v2_s
            return t_s, ithr

        res = []
        for j in range(rpw):
            res.append(do_row(wid * rpw + j))

        for p in range(nst):
            stage_v[pl.ds(p * 16, 16)] = jnp.where(
                iota16 < 8, res[2 * p][0], res[2 * p + 1][0]
            )
            stage_v[pl.ds((nst + p) * 16, 16)] = jnp.where(
                iota16 < 8, res[2 * p][1], res[2 * p + 1][1]
            )
        span = nst * 16
        pltpu.sync_copy(stage_v.at[pl.ds(0, span)],
                        tkey_hbm.at[pl.ds(wid * span, span)])
        pltpu.sync_copy(stage_v.at[pl.ds(span, span)],
                        ithr_hbm.at[pl.ds(wid * span, span)])

    run = pl.kernel(
        body,
        out_type=(
            jax.ShapeDtypeStruct((nrows * 8,), jnp.int32),
            jax.ShapeDtypeStruct((nrows * 8,), jnp.int32),
        ),
        mesh=mesh,
        compiler_params=pltpu.CompilerParams(needs_layout_passes=False),
        scratch_types=(
            pltpu.VMEM((_N,), jnp.int32),
            pltpu.VMEM((_NGRP,), jnp.int32),
            pltpu.VMEM((_NGRP + 16,), jnp.int32),
            pltpu.VMEM((_POOL,), jnp.int32),
            pltpu.VMEM((_POOL,), jnp.int32),
            pltpu.VMEM((_POOL,), jnp.int32),
            pltpu.VMEM((64,), jnp.int32),
        ),
    )
    return run(x_i32)


def _mask_kernel(x_ref, t_ref, i_ref, out_ref):
    k = _monokey_i(x_ref[...])
    t = t_ref[:, 0:1]
    it_ = i_ref[:, 0:1]
    col = lax.broadcasted_iota(jnp.int32, k.shape, 1)
    mask = ((k > t) | ((k == t) & (col <= it_))).astype(jnp.float32)
    out_ref[...] = jnp.broadcast_to(mask[None], (_NUM_SAMPLES,) + mask.shape)


def _mask_kernel_chain(x_ref, t_ref, i_ref, prev_ref, out_ref):
    del prev_ref
    _mask_kernel(x_ref, t_ref, i_ref, out_ref)


_CHUNKS = 2


def kernel(logits, tau):
    B, N, _ = logits.shape
    x_i32 = jax.lax.bitcast_convert_type(jnp.squeeze(logits, -1), jnp.int32)
    cb = B // _CHUNKS
    parts = [
        _sc_select(lax.slice_in_dim(x_i32, c * cb, (c + 1) * cb, axis=0), cb)
        for c in range(_CHUNKS)
    ]
    grid = (cb // _ROW_BLOCK,)
    out = None
    for c in range(_CHUNKS):
        tkey2d = parts[c][0].reshape(cb, 8)
        ithr2d = parts[c][1].reshape(cb, 8)
        b = c * (cb // _ROW_BLOCK)
        in_specs = [
            pl.BlockSpec((_ROW_BLOCK, N), lambda i, b=b: (i + b, 0)),
            pl.BlockSpec((_ROW_BLOCK, 8), lambda i: (i, 0)),
            pl.BlockSpec((_ROW_BLOCK, 8), lambda i: (i, 0)),
        ]
        args = [x_i32, tkey2d, ithr2d]
        kwargs = {}
        body_fn = _mask_kernel
        if c > 0:
            body_fn = _mask_kernel_chain
            in_specs.append(pl.BlockSpec(memory_space=pl.MemorySpace.ANY))
            args.append(out)
            kwargs = dict(input_output_aliases={3: 0})
        out = pl.pallas_call(
            body_fn,
            grid=grid,
            in_specs=in_specs,
            out_specs=pl.BlockSpec(
                (_NUM_SAMPLES, _ROW_BLOCK, N), lambda i, b=b: (0, i + b, 0)
            ),
            out_shape=jax.ShapeDtypeStruct((_NUM_SAMPLES, B, N), jnp.float32),
            **kwargs,
        )(*args)
    return out[..., None]


# speedup vs baseline: 1.0247x; 1.0247x over previous
import jax
import jax.numpy as jnp
from jax import lax
from jax.experimental import pallas as pl
from jax.experimental.pallas import tpu as pltpu
from jax.experimental.pallas import tpu_sc as plsc

_K = 64
_NUM_SAMPLES = 4
_B = 128
_N = 32768
_ROW_BLOCK = 8

_INT_MIN = -(2 ** 31)
_NW = 32
_RPW = _B // _NW
_NBLK = 32
_BLKV = 64
_NGRP = _NBLK * 16
_POOL = 4096


def _monokey(x):
    b = jax.lax.bitcast_convert_type(x, jnp.int32)
    return b ^ ((b >> 31) & jnp.int32(0x7FFFFFFF))


def _monokey_i(b):
    return b ^ ((b >> 31) & jnp.int32(0x7FFFFFFF))


def _count16(mask):
    return plsc.cumsum(mask.astype(jnp.int32))[15]


def _radix_select(ref, nv, target, start_bit, nbits, init_prefix):
    def bitbody(i, prefix_u):
        b = start_bit - i
        cand_u = prefix_u | lax.shift_left(jnp.int32(1), b)
        cand_s = cand_u ^ _INT_MIN

        def cntbody(v, acc):
            kk = ref[pl.ds(v * 16, 16)]
            return acc + _count16(kk >= cand_s)

        cnt = lax.fori_loop(0, nv, cntbody, jnp.int32(0))
        return jnp.where(cnt >= target, cand_u, prefix_u)

    prefix_u = lax.fori_loop(0, nbits, bitbody, init_prefix)
    return prefix_u ^ _INT_MIN


def _sc_select(x_i32, nrows):
    rpw = nrows // _NW
    nst = rpw // 2
    mesh = plsc.VectorSubcoreMesh(
        core_axis_name="c", subcore_axis_name="s", num_cores=2, num_subcores=16
    )

    def body(x_hbm, tkey_hbm, ithr_hbm, row_v, kmax_v, qids_v, poolk_v,
             pooli_v, keys2_v, stage_v):
        wid = lax.axis_index("c") * 16 + lax.axis_index("s")
        iota16 = lax.iota(jnp.int32, 16)
        iota16x = iota16 * 16

        def do_row(row):
            pltpu.sync_copy(x_hbm.at[row], row_v)

            def blkbody(ob, unused):
                base0 = ob * 1024

                def inner(it, m):
                    base = base0 + it * 128
                    for jj in range(8):
                        m = jnp.maximum(
                            m, _monokey_i(row_v[pl.ds(base + jj * 16, 16)])
                        )
                    return m

                m = lax.fori_loop(
                    0, 8, inner, jnp.full((16,), _INT_MIN, jnp.int32)
                )
                kmax_v[pl.ds(ob * 16, 16)] = m
                return unused

            lax.fori_loop(0, _NBLK, blkbody, jnp.int32(0))

            tau_s = _radix_select(kmax_v, _NGRP // 16, _K, 31, 16, jnp.int32(0))

            def qbody(i, qoff):
                kk = kmax_v[pl.ds(i * 16, 16)]
                m = kk >= tau_s
                ci = plsc.cumsum(m.astype(jnp.int32))
                pos = qoff + ci - 1
                plsc.store_scatter(qids_v, [pos], i * 16 + iota16, mask=m)
                return qoff + ci[15]

            nq = lax.fori_loop(0, _NGRP // 16, qbody, jnp.int32(0))

            def gbody(q, poff):
                gid = qids_v[pl.ds(q, 16)][0]
                base = (gid >> 4) * 1024 + (gid & 15)
                offs = jnp.int32(0)
                for jj in range(4):
                    idx = base + jj * 256 + iota16x
                    kv = _monokey_i(plsc.load_gather(row_v, [idx]))
                    m = kv >= tau_s
                    ci = plsc.cumsum(m.astype(jnp.int32))
                    pos = jnp.minimum(poff + offs + ci - 1,
                                      jnp.int32(_POOL - 1))
                    plsc.store_scatter(poolk_v, [pos], kv, mask=m)
                    plsc.store_scatter(pooli_v, [pos], idx, mask=m)
                    offs = offs + ci[15]
                return jnp.minimum(poff + offs, jnp.int32(_POOL - 16))

            poff = lax.fori_loop(0, nq, gbody, jnp.int32(0))

            plsc.store_scatter(
                poolk_v, [poff + iota16],
                jnp.full((16,), _INT_MIN, jnp.int32),
            )

            nv = (poff + 15) >> 4
            t_s = _radix_select(poolk_v, nv, _K, 31, 32, jnp.int32(0))

            def ebody(v, acc):
                kk = poolk_v[pl.ds(v * 16, 16)]
                acc = acc + _count16(kk > t_s)
                k2 = jnp.where(
                    kk == t_s, jnp.int32(_N - 1) - pooli_v[pl.ds(v * 16, 16)],
                    _INT_MIN,
                )
                keys2_v[pl.ds(v * 16, 16)] = k2
                return acc

            cnt_gt = lax.fori_loop(0, nv, ebody, jnp.int32(0))
            need = _K - cnt_gt
            v2_s = _radix_select(keys2_v, nv, need, 14, 15, jnp.int32(_INT_MIN))
            ithr = jnp.int32(_N - 1) - v2_s
            return t_s, ithr

        res = []
        for j in range(rpw):
            res.append(do_row(wid * rpw + j))

        for p in range(nst):
            stage_v[pl.ds(p * 16, 16)] = jnp.where(
                iota16 < 8, res[2 * p][0], res[2 * p + 1][0]
            )
            stage_v[pl.ds((nst + p) * 16, 16)] = jnp.where(
                iota16 < 8, res[2 * p][1], res[2 * p + 1][1]
            )
        span = nst * 16
        pltpu.sync_copy(stage_v.at[pl.ds(0, span)],
                        tkey_hbm.at[pl.ds(wid * span, span)])
        pltpu.sync_copy(stage_v.at[pl.ds(span, span)],
                        ithr_hbm.at[pl.ds(wid * span, span)])

    run = pl.kernel(
        body,
        out_type=(
            jax.ShapeDtypeStruct((nrows * 8,), jnp.int32),
            jax.ShapeDtypeStruct((nrows * 8,), jnp.int32),
        ),
        mesh=mesh,
        compiler_params=pltpu.CompilerParams(needs_layout_passes=False),
        scratch_types=(
            pltpu.VMEM((_N,), jnp.int32),
            pltpu.VMEM((_NGRP,), jnp.int32),
            pltpu.VMEM((_NGRP + 16,), jnp.int32),
            pltpu.VMEM((_POOL,), jnp.int32),
            pltpu.VMEM((_POOL,), jnp.int32),
            pltpu.VMEM((_POOL,), jnp.int32),
            pltpu.VMEM((64,), jnp.int32),
        ),
    )
    return run(x_i32)


def _mask_kernel(x_ref, t_ref, i_ref, out_ref):
    k = _monokey_i(x_ref[...])
    t = t_ref[:, 0:1]
    it_ = i_ref[:, 0:1]
    col = lax.broadcasted_iota(jnp.int32, k.shape, 1)
    out_ref[...] = ((k > t) | ((k == t) & (col <= it_))).astype(jnp.float32)


def kernel(logits, tau):
    B, N, _ = logits.shape
    x_i32 = jax.lax.bitcast_convert_type(jnp.squeeze(logits, -1), jnp.int32)
    tkey, ithr = _sc_select(x_i32, B)
    tkey2d = tkey.reshape(B, 8)
    ithr2d = ithr.reshape(B, 8)
    grid = (B // _ROW_BLOCK,)
    mask = pl.pallas_call(
        _mask_kernel,
        grid=grid,
        in_specs=[
            pl.BlockSpec((_ROW_BLOCK, N), lambda i: (i, 0)),
            pl.BlockSpec((_ROW_BLOCK, 8), lambda i: (i, 0)),
            pl.BlockSpec((_ROW_BLOCK, 8), lambda i: (i, 0)),
        ],
        out_specs=pl.BlockSpec((_ROW_BLOCK, N), lambda i: (i, 0)),
        out_shape=jax.ShapeDtypeStruct((B, N), jnp.float32),
    )(x_i32, tkey2d, ithr2d)
    return jnp.broadcast_to(mask[None, :, :, None], (_NUM_SAMPLES, B, N, 1))


# speedup vs baseline: 1.0671x; 1.0414x over previous
import jax
import jax.numpy as jnp
from jax import lax
from jax.experimental import pallas as pl
from jax.experimental.pallas import tpu as pltpu
from jax.experimental.pallas import tpu_sc as plsc

_K = 64
_NUM_SAMPLES = 4
_B = 128
_N = 32768
_ROW_BLOCK = 8

_INT_MIN = -(2 ** 31)
_NW = 32
_RPW = _B // _NW
_NBLK = 32
_BLKV = 64
_NGRP = _NBLK * 16
_POOL = 4096


def _monokey(x):
    b = jax.lax.bitcast_convert_type(x, jnp.int32)
    return b ^ ((b >> 31) & jnp.int32(0x7FFFFFFF))


def _monokey_i(b):
    return b ^ ((b >> 31) & jnp.int32(0x7FFFFFFF))


def _count16(mask):
    return plsc.cumsum(mask.astype(jnp.int32))[15]


def _radix_select(ref, nv, target, start_bit, nbits, init_prefix):
    def bitbody(i, prefix_u):
        b = start_bit - i
        cand_u = prefix_u | lax.shift_left(jnp.int32(1), b)
        cand_s = cand_u ^ _INT_MIN

        def cntbody(v, acc):
            kk = ref[pl.ds(v * 16, 16)]
            return acc + _count16(kk >= cand_s)

        cnt = lax.fori_loop(0, nv, cntbody, jnp.int32(0))
        return jnp.where(cnt >= target, cand_u, prefix_u)

    prefix_u = lax.fori_loop(0, nbits, bitbody, init_prefix)
    return prefix_u ^ _INT_MIN


def _sc_select(x_i32, nrows):
    rpw = nrows // _NW
    nst = rpw // 2
    mesh = plsc.VectorSubcoreMesh(
        core_axis_name="c", subcore_axis_name="s", num_cores=2, num_subcores=16
    )

    def body(x_hbm, tkey_hbm, ithr_hbm, row_v, kmax_v, qids_v, poolk_v,
             pooli_v, keys2_v, stage_v):
        wid = lax.axis_index("c") * 16 + lax.axis_index("s")
        iota16 = lax.iota(jnp.int32, 16)
        iota16x = iota16 * 16

        def do_row(row):
            pltpu.sync_copy(x_hbm.at[row], row_v)

            def blkbody(ob, unused):
                base0 = ob * 1024

                def inner(it, m):
                    base = base0 + it * 128
                    for jj in range(8):
                        m = jnp.maximum(
                            m, _monokey_i(row_v[pl.ds(base + jj * 16, 16)])
                        )
                    return m

                m = lax.fori_loop(
                    0, 8, inner, jnp.full((16,), _INT_MIN, jnp.int32)
                )
                kmax_v[pl.ds(ob * 16, 16)] = m
                return unused

            lax.fori_loop(0, _NBLK, blkbody, jnp.int32(0))

            tau_s = _radix_select(kmax_v, _NGRP // 16, _K, 31, 16, jnp.int32(0))

            def qbody(i, qoff):
                kk = kmax_v[pl.ds(i * 16, 16)]
                m = kk >= tau_s
                ci = plsc.cumsum(m.astype(jnp.int32))
                pos = qoff + ci - 1
                plsc.store_scatter(qids_v, [pos], i * 16 + iota16, mask=m)
                return qoff + ci[15]

            nq = lax.fori_loop(0, _NGRP // 16, qbody, jnp.int32(0))

            def gbody(q, poff):
                gid = qids_v[pl.ds(q, 16)][0]
                base = (gid >> 4) * 1024 + (gid & 15)
                offs = jnp.int32(0)
                for jj in range(4):
                    idx = base + jj * 256 + iota16x
                    kv = _monokey_i(plsc.load_gather(row_v, [idx]))
                    m = kv >= tau_s
                    ci = plsc.cumsum(m.astype(jnp.int32))
                    pos = jnp.minimum(poff + offs + ci - 1,
                                      jnp.int32(_POOL - 1))
                    plsc.store_scatter(poolk_v, [pos], kv, mask=m)
                    plsc.store_scatter(pooli_v, [pos], idx, mask=m)
                    offs = offs + ci[15]
                return jnp.minimum(poff + offs, jnp.int32(_POOL - 16))

            poff = lax.fori_loop(0, nq, gbody, jnp.int32(0))

            plsc.store_scatter(
                poolk_v, [poff + iota16],
                jnp.full((16,), _INT_MIN, jnp.int32),
            )

            nv = (poff + 15) >> 4
            t_s = _radix_select(poolk_v, nv, _K, 31, 32, jnp.int32(0))

            def ebody(v, acc):
                kk = poolk_v[pl.ds(v * 16, 16)]
                acc = acc + _count16(kk > t_s)
                k2 = jnp.where(
                    kk == t_s, jnp.int32(_N - 1) - pooli_v[pl.ds(v * 16, 16)],
                    _INT_MIN,
                )
                keys2_v[pl.ds(v * 16, 16)] = k2
                return acc

            cnt_gt = lax.fori_loop(0, nv, ebody, jnp.int32(0))
            need = _K - cnt_gt
            v2_s = _radix_select(keys2_v, nv, need, 14, 15, jnp.int32(_INT_MIN))
            ithr = jnp.int32(_N - 1) - v2_s
            return t_s, ithr

        res = []
        for j in range(rpw):
            res.append(do_row(wid * rpw + j))

        for p in range(nst):
            stage_v[pl.ds(p * 16, 16)] = jnp.where(
                iota16 < 8, res[2 * p][0], res[2 * p + 1][0]
            )
            stage_v[pl.ds((nst + p) * 16, 16)] = jnp.where(
                iota16 < 8, res[2 * p][1], res[2 * p + 1][1]
            )
        span = nst * 16
        pltpu.sync_copy(stage_v.at[pl.ds(0, span)],
                        tkey_hbm.at[pl.ds(wid * span, span)])
        pltpu.sync_copy(stage_v.at[pl.ds(span, span)],
                        ithr_hbm.at[pl.ds(wid * span, span)])

    run = pl.kernel(
        body,
        out_type=(
            jax.ShapeDtypeStruct((nrows * 8,), jnp.int32),
            jax.ShapeDtypeStruct((nrows * 8,), jnp.int32),
        ),
        mesh=mesh,
        compiler_params=pltpu.CompilerParams(needs_layout_passes=False),
        scratch_types=(
            pltpu.VMEM((_N,), jnp.int32),
            pltpu.VMEM((_NGRP,), jnp.int32),
            pltpu.VMEM((_NGRP + 16,), jnp.int32),
            pltpu.VMEM((_POOL,), jnp.int32),
            pltpu.VMEM((_POOL,), jnp.int32),
            pltpu.VMEM((_POOL,), jnp.int32),
            pltpu.VMEM((64,), jnp.int32),
        ),
    )
    return run(x_i32)


def _mask_kernel(x_ref, t_ref, i_ref, out_ref):
    k = _monokey_i(x_ref[...])
    t = t_ref[:, 0:1]
    it_ = i_ref[:, 0:1]
    col = lax.broadcasted_iota(jnp.int32, k.shape, 1)
    mask = ((k > t) | ((k == t) & (col <= it_))).astype(jnp.float32)
    out_ref[...] = jnp.broadcast_to(mask[None], (_NUM_SAMPLES,) + mask.shape)


def kernel(logits, tau):
    B, N, _ = logits.shape
    x_i32 = jax.lax.bitcast_convert_type(jnp.squeeze(logits, -1), jnp.int32)
    tkey, ithr = _sc_select(x_i32, B)
    tkey2d = tkey.reshape(B, 8)
    ithr2d = ithr.reshape(B, 8)
    grid = (B // _ROW_BLOCK,)
    out = pl.pallas_call(
        _mask_kernel,
        grid=grid,
        in_specs=[
            pl.BlockSpec((_ROW_BLOCK, N), lambda i: (i, 0)),
            pl.BlockSpec((_ROW_BLOCK, 8), lambda i: (i, 0)),
            pl.BlockSpec((_ROW_BLOCK, 8), lambda i: (i, 0)),
        ],
        out_specs=pl.BlockSpec((_NUM_SAMPLES, _ROW_BLOCK, N), lambda i: (0, i, 0)),
        out_shape=jax.ShapeDtypeStruct((_NUM_SAMPLES, B, N), jnp.float32),
    )(x_i32, tkey2d, ithr2d)
    return out[..., None]


# speedup vs baseline: 1.0685x; 1.0013x over previous
import jax
import jax.numpy as jnp
from jax import lax
from jax.experimental import pallas as pl
from jax.experimental.pallas import tpu as pltpu
from jax.experimental.pallas import tpu_sc as plsc

_K = 64
_NUM_SAMPLES = 4
_B = 128
_N = 32768
_ROW_BLOCK = 8

_INT_MIN = -(2 ** 31)
_NW = 32
_NBLK = 32
_NGRP = _NBLK * 16
_POOL = 4096


def _monokey_i(b):
    return b ^ ((b >> 31) & jnp.int32(0x7FFFFFFF))


def _count16(mask):
    return plsc.cumsum(mask.astype(jnp.int32))[15]


def _radix_select(ref, nv, target, start_bit, nbits, init_prefix):
    def bitbody(i, prefix_u):
        b = start_bit - i
        cand_u = prefix_u | lax.shift_left(jnp.int32(1), b)
        cand_s = cand_u ^ _INT_MIN

        def cntbody(v, acc):
            kk = ref[pl.ds(v * 16, 16)]
            return acc + _count16(kk >= cand_s)

        cnt = lax.fori_loop(0, nv, cntbody, jnp.int32(0))
        return jnp.where(cnt >= target, cand_u, prefix_u)

    prefix_u = lax.fori_loop(0, nbits, bitbody, init_prefix)
    return prefix_u ^ _INT_MIN


def _sc_select(x_i32, nrows):
    rpw = nrows // _NW
    nst = rpw // 2
    mesh = plsc.VectorSubcoreMesh(
        core_axis_name="c", subcore_axis_name="s", num_cores=2, num_subcores=16
    )

    def body(x_hbm, tkey_hbm, ithr_hbm, row_v, kmax_v, qids_v, poolk_v,
             pooli_v, keys2_v, stage_v):
        wid = lax.axis_index("c") * 16 + lax.axis_index("s")
        iota16 = lax.iota(jnp.int32, 16)
        iota16x = iota16 * 16

        def do_row(row):
            pltpu.sync_copy(x_hbm.at[row], row_v)

            def blkbody(ob, unused):
                base0 = ob * 1024

                def inner(it, m):
                    base = base0 + it * 128
                    for jj in range(8):
                        m = jnp.maximum(
                            m, _monokey_i(row_v[pl.ds(base + jj * 16, 16)])
                        )
                    return m

                m = lax.fori_loop(
                    0, 8, inner, jnp.full((16,), _INT_MIN, jnp.int32)
                )
                kmax_v[pl.ds(ob * 16, 16)] = m
                return unused

            lax.fori_loop(0, _NBLK, blkbody, jnp.int32(0))

            tau_s = _radix_select(kmax_v, _NGRP // 16, _K, 31, 16, jnp.int32(0))

            def qbody(i, qoff):
                kk = kmax_v[pl.ds(i * 16, 16)]
                m = kk >= tau_s
                ci = plsc.cumsum(m.astype(jnp.int32))
                pos = qoff + ci - 1
                plsc.store_scatter(qids_v, [pos], i * 16 + iota16, mask=m)
                return qoff + ci[15]

            nq = lax.fori_loop(0, _NGRP // 16, qbody, jnp.int32(0))

            def gbody(q, poff):
                gid = qids_v[pl.ds(q, 16)][0]
                base = (gid >> 4) * 1024 + (gid & 15)
                offs = jnp.int32(0)
                for jj in range(4):
                    idx = base + jj * 256 + iota16x
                    kv = _monokey_i(plsc.load_gather(row_v, [idx]))
                    m = kv >= tau_s
                    ci = plsc.cumsum(m.astype(jnp.int32))
                    pos = jnp.minimum(poff + offs + ci - 1,
                                      jnp.int32(_POOL - 1))
                    plsc.store_scatter(poolk_v, [pos], kv, mask=m)
                    plsc.store_scatter(pooli_v, [pos], idx, mask=m)
                    offs = offs + ci[15]
                return jnp.minimum(poff + offs, jnp.int32(_POOL - 16))

            poff = lax.fori_loop(0, nq, gbody, jnp.int32(0))

            plsc.store_scatter(
                poolk_v, [poff + iota16],
                jnp.full((16,), _INT_MIN, jnp.int32),
            )

            nv = (poff + 15) >> 4
            t_s = _radix_select(poolk_v, nv, _K, 31, 32, jnp.int32(0))

            def ebody(v, acc):
                kk = poolk_v[pl.ds(v * 16, 16)]
                acc = acc + _count16(kk > t_s)
                k2 = jnp.where(
                    kk == t_s, jnp.int32(_N - 1) - pooli_v[pl.ds(v * 16, 16)],
                    _INT_MIN,
                )
                keys2_v[pl.ds(v * 16, 16)] = k2
                return acc

            cnt_gt = lax.fori_loop(0, nv, ebody, jnp.int32(0))
            need = _K - cnt_gt
            v2_s = _radix_select(keys2_v, nv, need, 14, 15, jnp.int32(_INT_MIN))
            ithr = jnp.int32(_N - 1) - v2_s
            return t_s, ithr

        res = []
        for j in range(rpw):
            res.append(do_row(wid * rpw + j))

        for p in range(nst):
            stage_v[pl.ds(p * 16, 16)] = jnp.where(
                iota16 < 8, res[2 * p][0], res[2 * p + 1][0]
            )
            stage_v[pl.ds((nst + p) * 16, 16)] = jnp.where(
                iota16 < 8, res[2 * p][1], res[2 * p + 1][1]
            )
        span = nst * 16
        pltpu.sync_copy(stage_v.at[pl.ds(0, span)],
                        tkey_hbm.at[pl.ds(wid * span, span)])
        pltpu.sync_copy(stage_v.at[pl.ds(span, span)],
                        ithr_hbm.at[pl.ds(wid * span, span)])

    run = pl.kernel(
        body,
        out_type=(
            jax.ShapeDtypeStruct((nrows * 8,), jnp.int32),
            jax.ShapeDtypeStruct((nrows * 8,), jnp.int32),
        ),
        mesh=mesh,
        compiler_params=pltpu.CompilerParams(needs_layout_passes=False),
        scratch_types=(
            pltpu.VMEM((_N,), jnp.int32),
            pltpu.VMEM((_NGRP,), jnp.int32),
            pltpu.VMEM((_NGRP + 16,), jnp.int32),
            pltpu.VMEM((_POOL,), jnp.int32),
            pltpu.VMEM((_POOL,), jnp.int32),
            pltpu.VMEM((_POOL,), jnp.int32),
            pltpu.VMEM((64,), jnp.int32),
        ),
    )
    return run(x_i32)


def _mask_kernel(x_ref, t_ref, i_ref, out_ref):
    k = _monokey_i(x_ref[...])
    t = t_ref[:, 0:1]
    it_ = i_ref[:, 0:1]
    col = lax.broadcasted_iota(jnp.int32, k.shape, 1)
    mask = ((k > t) | ((k == t) & (col <= it_))).astype(jnp.float32)
    out_ref[...] = jnp.broadcast_to(mask[None], (_NUM_SAMPLES,) + mask.shape)


def kernel(logits, tau):
    B, N, _ = logits.shape
    x_i32 = jax.lax.bitcast_convert_type(jnp.squeeze(logits, -1), jnp.int32)
    tkey, ithr = _sc_select(x_i32, B)
    tkey2d = tkey.reshape(B, 8)
    ithr2d = ithr.reshape(B, 8)
    grid = (B // _ROW_BLOCK,)
    out = pl.pallas_call(
        _mask_kernel,
        grid=grid,
        in_specs=[
            pl.BlockSpec((_ROW_BLOCK, N), lambda i: (i, 0)),
            pl.BlockSpec((_ROW_BLOCK, 8), lambda i: (i, 0)),
            pl.BlockSpec((_ROW_BLOCK, 8), lambda i: (i, 0)),
        ],
        out_specs=pl.BlockSpec((_NUM_SAMPLES, _ROW_BLOCK, N), lambda i: (0, i, 0)),
        out_shape=jax.ShapeDtypeStruct((_NUM_SAMPLES, B, N), jnp.float32),
    )(x_i32, tkey2d, ithr2d)
    return out[..., None]
